# baseline (speedup 1.0000x reference)
import functools

import jax
import jax.numpy as jnp
from jax import lax
from jax.experimental import pallas as pl
from jax.experimental.pallas import tpu as pltpu
from jax.experimental.pallas import tpu_sc as plsc

_D_MAIN = 49152
_D_PAWN = 8192
_F_MAIN = 768
_F_PAWN = 128
_BASE = 160
_P_BASE = 256
_N = 4096

_NC = 2
_NS = 16
_NW = _NC * _NS
_SW = _N // _NW
_KM = 32
_KP = 8
_CS_M = 128 // _KM
_CS_P = 128 // _KP
_NCH_M = _SW // _CS_M
_NCH_P = _SW // _CS_P


def _sc_body(widx, bidx, pwidx, pbidx, wt, bt, pwt, pbt,
             wout, bout, pwout, pbout,
             idx_mw, idx_mb, idx_pw, idx_pb, blk_m, blk_p,
             acc_mw, acc_mb, acc_pw, acc_pb, sem):
    wid = lax.axis_index("s") * _NC + lax.axis_index("c")
    base = pl.ds(wid * _SW, _SW)
    lanes = lax.iota(jnp.int32, 16)

    def stage_t(idx_hbm, blk, dst, k):
        pltpu.sync_copy(idx_hbm.at[base], blk)
        for g in range(_SW // 16):
            rows = lanes + g * 16
            for j in range(k):
                v = plsc.load_gather(blk, [rows, jnp.full((16,), j, jnp.int32)])
                dst[j, pl.ds(g * 16, 16)] = v

    stage_t(widx, blk_m, idx_mw, _KM)
    stage_t(bidx, blk_m, idx_mb, _KM)
    stage_t(pwidx, blk_p, idx_pw, _KP)
    stage_t(pbidx, blk_p, idx_pb, _KP)

    pltpu.async_copy(wt.at[idx_mw.at[0]], acc_mw, sem).wait()
    pltpu.async_copy(bt.at[idx_mb.at[0]], acc_mb, sem).wait()
    pltpu.async_copy(pwt.at[idx_pw.at[0]], acc_pw, sem).wait()
    pltpu.async_copy(pbt.at[idx_pb.at[0]], acc_pb, sem).wait()

    descs = []
    for j in range(1, _KM):
        descs.append(pltpu.async_copy(wt.at[idx_mw.at[j]], acc_mw, sem,
                                      add=True))
        descs.append(pltpu.async_copy(bt.at[idx_mb.at[j]], acc_mb, sem,
                                      add=True))
    for j in range(1, _KP):
        descs.append(pltpu.async_copy(pwt.at[idx_pw.at[j]], acc_pw, sem,
                                      add=True))
        descs.append(pltpu.async_copy(pbt.at[idx_pb.at[j]], acc_pb, sem,
                                      add=True))
    for d in descs:
        d.wait()

    pltpu.sync_copy(acc_mw, wout.at[base])
    pltpu.sync_copy(acc_mb, bout.at[base])
    pltpu.sync_copy(acc_pw, pwout.at[base])
    pltpu.sync_copy(acc_pb, pbout.at[base])


@functools.cache
def _sc_lookup_fn():
    return pl.kernel(
        _sc_body,
        out_type=(
            jax.ShapeDtypeStruct((_N, _BASE), jnp.float32),
            jax.ShapeDtypeStruct((_N, _BASE), jnp.float32),
            jax.ShapeDtypeStruct((_N, _P_BASE), jnp.float32),
            jax.ShapeDtypeStruct((_N, _P_BASE), jnp.float32),
        ),
        mesh=plsc.VectorSubcoreMesh(core_axis_name="c", subcore_axis_name="s",
                                    num_cores=_NC, num_subcores=_NS),
        scratch_types=[
            pltpu.VMEM((_KM, _SW), jnp.int32),
            pltpu.VMEM((_KM, _SW), jnp.int32),
            pltpu.VMEM((_KP, _SW), jnp.int32),
            pltpu.VMEM((_KP, _SW), jnp.int32),
            pltpu.VMEM((_SW, _KM), jnp.int32),
            pltpu.VMEM((_SW, _KP), jnp.int32),
            pltpu.VMEM((_SW, _BASE), jnp.float32),
            pltpu.VMEM((_SW, _BASE), jnp.float32),
            pltpu.VMEM((_SW, _P_BASE), jnp.float32),
            pltpu.VMEM((_SW, _P_BASE), jnp.float32),
            pltpu.SemaphoreType.DMA,
        ],
        compiler_params=pltpu.CompilerParams(use_tc_tiling_on_sc=False,
                                             needs_layout_passes=False),
    )


def _fold_add_body(aff, factt, out):
    out[...] = aff[...] + factt[...]


def _fold_table(aff, factt, fb):
    b, d = aff.shape
    return pl.pallas_call(
        _fold_add_body,
        grid=(d // fb,),
        in_specs=[pl.BlockSpec((b, fb), lambda j: (0, j)),
                  pl.BlockSpec((b, fb), lambda j: (0, 0))],
        out_specs=pl.BlockSpec((b, fb), lambda j: (0, j)),
        out_shape=jax.ShapeDtypeStruct((b, d), jnp.float32),
    )(aff, factt)


_BN = 1024


def _head_body(pov, w, b, pw, pb, wbias, bbias, pwbias, pbbias,
               f0a, f0b, f0c, p0a, p0b, p0c, p1t, p1c, f1t, f1c,
               f2a, f2b, f2c, f3a, f3b, f3c, f3d, out):
    q = pov[...]
    r = 1.0 - q
    wv = w[...] + wbias[...]
    bv = b[...] + bbias[...]
    pwv = pw[...] + pwbias[...]
    pbv = pb[...] + pbbias[...]
    base1 = jnp.maximum(q * wv + r * bv, 0.0)
    base2 = jnp.maximum(q * bv + r * wv, 0.0)
    pb1 = jnp.maximum(q * pwv + r * pbv, 0.0)
    pb2 = jnp.maximum(q * pbv + r * pwv, 0.0)
    dot = functools.partial(jnp.dot, preferred_element_type=jnp.float32)
    p = jnp.maximum(dot(pb1, p0a[...]) + dot(pb2, p0b[...]) + p0c[...], 0.0)
    p = dot(p, p1t[...]) + p1c[...]
    x = jnp.maximum(dot(base1, f0a[...]) + dot(base2, f0b[...]) + f0c[...] + p,
                    0.0)
    y = jnp.maximum(dot(x, f1t[...]) + f1c[...], 0.0)
    z = jnp.maximum(dot(x, f2a[...]) + dot(y, f2b[...]) + f2c[...], 0.0)
    out[...] = dot(x, f3a[...]) + dot(y, f3b[...]) + dot(z, f3c[...]) + f3d[...]


def _row_spec(cols):
    return pl.BlockSpec((_BN, cols), lambda i: (i, 0))


def _full_spec(rows, cols):
    return pl.BlockSpec((rows, cols), lambda i: (0, 0))


def kernel(pov, white_idx, black_idx, p_white_idx, p_black_idx,
           w_aff_W, w_aff_b, w_fact_W, b_aff_W, b_aff_b, b_fact_W,
           pw_aff_W, pw_aff_b, pw_fact_W, pb_aff_W, pb_aff_b, pb_fact_W,
           fc0_W, fc0_b, p_fc0_W, p_fc0_b, p_fc1_W, p_fc1_b,
           fc1_W, fc1_b, fc2_W, fc2_b, fc3_W, fc3_b):
    w_comb = _fold_table(w_aff_W, w_fact_W.T, _F_MAIN).T
    b_comb = _fold_table(b_aff_W, b_fact_W.T, _F_MAIN).T
    pw_comb = _fold_table(pw_aff_W, jnp.tile(pw_fact_W.T, (1, 4)),
                          4 * _F_PAWN).T
    pb_comb = _fold_table(pb_aff_W, jnp.tile(pb_fact_W.T, (1, 4)),
                          4 * _F_PAWN).T

    wsum, bsum, pwsum, pbsum = _sc_lookup_fn()(
        white_idx, black_idx, p_white_idx, p_black_idx,
        w_comb, b_comb, pw_comb, pb_comb)

    grid = (_N // _BN,)
    out = pl.pallas_call(
        _head_body,
        grid=grid,
        in_specs=[
            _row_spec(1),
            _row_spec(_BASE), _row_spec(_BASE),
            _row_spec(_P_BASE), _row_spec(_P_BASE),
            _full_spec(1, _BASE), _full_spec(1, _BASE),
            _full_spec(1, _P_BASE), _full_spec(1, _P_BASE),
            _full_spec(_BASE, 16), _full_spec(_BASE, 16), _full_spec(1, 16),
            _full_spec(_P_BASE, 16), _full_spec(_P_BASE, 16), _full_spec(1, 16),
            _full_spec(16, 16), _full_spec(1, 16),
            _full_spec(16, 16), _full_spec(1, 16),
            _full_spec(16, 16), _full_spec(16, 16), _full_spec(1, 16),
            _full_spec(16, 1), _full_spec(16, 1), _full_spec(16, 1),
            _full_spec(1, 1),
        ],
        out_specs=_row_spec(1),
        out_shape=jax.ShapeDtypeStruct((_N, 1), jnp.float32),
    )(
        pov, wsum, bsum, pwsum, pbsum,
        w_aff_b.reshape(1, -1), b_aff_b.reshape(1, -1),
        pw_aff_b.reshape(1, -1), pb_aff_b.reshape(1, -1),
        fc0_W[:, :_BASE].T, fc0_W[:, _BASE:].T, fc0_b.reshape(1, -1),
        p_fc0_W[:, :_P_BASE].T, p_fc0_W[:, _P_BASE:].T, p_fc0_b.reshape(1, -1),
        p_fc1_W.T, p_fc1_b.reshape(1, -1),
        fc1_W.T, fc1_b.reshape(1, -1),
        fc2_W[:, :16].T, fc2_W[:, 16:].T, fc2_b.reshape(1, -1),
        fc3_W[:, :16].T, fc3_W[:, 16:32].T, fc3_W[:, 32:].T,
        fc3_b.reshape(1, -1),
    )
    return out


# speedup vs baseline: 1.2584x; 1.2584x over previous
import functools

import jax
import jax.numpy as jnp
from jax import lax
from jax.experimental import pallas as pl
from jax.experimental.pallas import tpu as pltpu
from jax.experimental.pallas import tpu_sc as plsc

_D_MAIN = 49152
_D_PAWN = 8192
_F_MAIN = 768
_F_PAWN = 128
_BASE = 160
_P_BASE = 256
_N = 4096

_NC = 2
_NS = 16
_NW = _NC * _NS
_SW = _N // _NW
_KM = 32
_KP = 8


def _color_body(k):
    def body(idx_hbm, tab, out, blk, idxv, acc, sem):
        wid = lax.axis_index("s") * _NC + lax.axis_index("c")
        lanes = lax.iota(jnp.int32, 16)
        pltpu.sync_copy(idx_hbm.at[pl.ds(wid * _SW, _SW)], blk)
        for g in range(_SW // 16):
            rows = lanes + g * 16
            for j in range(k):
                v = plsc.load_gather(
                    blk, [rows, jnp.full((16,), j, jnp.int32)])
                idxv[j, pl.ds(g * 16, 16)] = v
        pltpu.async_copy(tab.at[idxv.at[0]], acc, sem).wait()
        descs = [pltpu.async_copy(tab.at[idxv.at[j]], acc, sem, add=True)
                 for j in range(1, k)]
        for d in descs:
            d.wait()
        pltpu.sync_copy(acc, out.at[pl.ds(wid * _SW, _SW)])
    return body


_color_kernels = {}


def _color_kernel(k, dout):
    key = (k, dout)
    if key not in _color_kernels:
        _color_kernels[key] = pl.kernel(
            _color_body(k),
            out_type=jax.ShapeDtypeStruct((_N, dout), jnp.float32),
            mesh=plsc.VectorSubcoreMesh(core_axis_name="c",
                                        subcore_axis_name="s",
                                        num_cores=_NC, num_subcores=_NS),
            scratch_types=[
                pltpu.VMEM((_SW, k), jnp.int32),
                pltpu.VMEM((k, _SW), jnp.int32),
                pltpu.VMEM((_SW, dout), jnp.float32),
                pltpu.SemaphoreType.DMA,
            ],
            compiler_params=pltpu.CompilerParams(use_tc_tiling_on_sc=False,
                                                 needs_layout_passes=False),
        )
    return _color_kernels[key]


def _fold_t_body(aff, factt, out):
    out[...] = (aff[...] + factt[...]).T


def _fold_table(aff, factt, fb):
    b, d = aff.shape
    return pl.pallas_call(
        _fold_t_body,
        grid=(d // fb,),
        in_specs=[pl.BlockSpec((b, fb), lambda j: (0, j)),
                  pl.BlockSpec((b, fb), lambda j: (0, 0))],
        out_specs=pl.BlockSpec((fb, b), lambda j: (j, 0)),
        out_shape=jax.ShapeDtypeStruct((d, b), jnp.float32),
    )(aff, factt)


_BN = 1024


def _head_body(pov, w, b, pw, pb, wbias, bbias, pwbias, pbbias,
               f0a, f0b, f0c, p0a, p0b, p0c, p1t, p1c, f1t, f1c,
               f2a, f2b, f2c, f3a, f3b, f3c, f3d, out):
    q = pov[...]
    r = 1.0 - q
    wv = w[...] + wbias[...]
    bv = b[...] + bbias[...]
    pwv = pw[...] + pwbias[...]
    pbv = pb[...] + pbbias[...]
    base1 = jnp.maximum(q * wv + r * bv, 0.0)
    base2 = jnp.maximum(q * bv + r * wv, 0.0)
    pb1 = jnp.maximum(q * pwv + r * pbv, 0.0)
    pb2 = jnp.maximum(q * pbv + r * pwv, 0.0)
    dot = functools.partial(jnp.dot, preferred_element_type=jnp.float32)
    p = jnp.maximum(dot(pb1, p0a[...]) + dot(pb2, p0b[...]) + p0c[...], 0.0)
    p = dot(p, p1t[...]) + p1c[...]
    x = jnp.maximum(dot(base1, f0a[...]) + dot(base2, f0b[...]) + f0c[...] + p,
                    0.0)
    y = jnp.maximum(dot(x, f1t[...]) + f1c[...], 0.0)
    z = jnp.maximum(dot(x, f2a[...]) + dot(y, f2b[...]) + f2c[...], 0.0)
    out[...] = dot(x, f3a[...]) + dot(y, f3b[...]) + dot(z, f3c[...]) + f3d[...]


def _row_spec(cols):
    return pl.BlockSpec((_BN, cols), lambda i: (i, 0))


def _full_spec(rows, cols):
    return pl.BlockSpec((rows, cols), lambda i: (0, 0))


def kernel(pov, white_idx, black_idx, p_white_idx, p_black_idx,
           w_aff_W, w_aff_b, w_fact_W, b_aff_W, b_aff_b, b_fact_W,
           pw_aff_W, pw_aff_b, pw_fact_W, pb_aff_W, pb_aff_b, pb_fact_W,
           fc0_W, fc0_b, p_fc0_W, p_fc0_b, p_fc1_W, p_fc1_b,
           fc1_W, fc1_b, fc2_W, fc2_b, fc3_W, fc3_b):
    pw_comb = _fold_table(pw_aff_W, jnp.tile(pw_fact_W.T, (1, 4)), 4 * _F_PAWN)
    pb_comb = _fold_table(pb_aff_W, jnp.tile(pb_fact_W.T, (1, 4)), 4 * _F_PAWN)
    w_comb = _fold_table(w_aff_W, w_fact_W.T, _F_MAIN)
    b_comb = _fold_table(b_aff_W, b_fact_W.T, _F_MAIN)

    pwsum = _color_kernel(_KP, _P_BASE)(p_white_idx, pw_comb)
    pbsum = _color_kernel(_KP, _P_BASE)(p_black_idx, pb_comb)
    wsum = _color_kernel(_KM, _BASE)(white_idx, w_comb)
    bsum = _color_kernel(_KM, _BASE)(black_idx, b_comb)

    grid = (_N // _BN,)
    out = pl.pallas_call(
        _head_body,
        grid=grid,
        in_specs=[
            _row_spec(1),
            _row_spec(_BASE), _row_spec(_BASE),
            _row_spec(_P_BASE), _row_spec(_P_BASE),
            _full_spec(1, _BASE), _full_spec(1, _BASE),
            _full_spec(1, _P_BASE), _full_spec(1, _P_BASE),
            _full_spec(_BASE, 16), _full_spec(_BASE, 16), _full_spec(1, 16),
            _full_spec(_P_BASE, 16), _full_spec(_P_BASE, 16), _full_spec(1, 16),
            _full_spec(16, 16), _full_spec(1, 16),
            _full_spec(16, 16), _full_spec(1, 16),
            _full_spec(16, 16), _full_spec(16, 16), _full_spec(1, 16),
            _full_spec(16, 1), _full_spec(16, 1), _full_spec(16, 1),
            _full_spec(1, 1),
        ],
        out_specs=_row_spec(1),
        out_shape=jax.ShapeDtypeStruct((_N, 1), jnp.float32),
    )(
        pov, wsum, bsum, pwsum, pbsum,
        w_aff_b.reshape(1, -1), b_aff_b.reshape(1, -1),
        pw_aff_b.reshape(1, -1), pb_aff_b.reshape(1, -1),
        fc0_W[:, :_BASE].T, fc0_W[:, _BASE:].T, fc0_b.reshape(1, -1),
        p_fc0_W[:, :_P_BASE].T, p_fc0_W[:, _P_BASE:].T, p_fc0_b.reshape(1, -1),
        p_fc1_W.T, p_fc1_b.reshape(1, -1),
        fc1_W.T, fc1_b.reshape(1, -1),
        fc2_W[:, :16].T, fc2_W[:, 16:].T, fc2_b.reshape(1, -1),
        fc3_W[:, :16].T, fc3_W[:, 16:32].T, fc3_W[:, 32:].T,
        fc3_b.reshape(1, -1),
    )
    return out


# speedup vs baseline: 1.2847x; 1.0209x over previous
import functools

import jax
import jax.numpy as jnp
from jax import lax
from jax.experimental import pallas as pl
from jax.experimental.pallas import tpu as pltpu
from jax.experimental.pallas import tpu_sc as plsc

_D_MAIN = 49152
_D_PAWN = 8192
_F_MAIN = 768
_F_PAWN = 128
_BASE = 160
_P_BASE = 256
_N = 4096

_NC = 2
_NS = 16
_NW = _NC * _NS
_SW = _N // _NW
_KM = 32
_KP = 8


def _color_body(k):
    def body(idx_hbm, tab, out, blk, idxv, acc, sem):
        wid = lax.axis_index("s") * _NC + lax.axis_index("c")
        lanes = lax.iota(jnp.int32, 16)
        pltpu.sync_copy(idx_hbm.at[pl.ds(wid * _SW, _SW)], blk)
        for g in range(_SW // 16):
            rows = lanes + g * 16
            for j in range(k):
                v = plsc.load_gather(
                    blk, [rows, jnp.full((16,), j, jnp.int32)])
                idxv[j, pl.ds(g * 16, 16)] = v
        pltpu.async_copy(tab.at[idxv.at[0]], acc, sem).wait()
        descs = [pltpu.async_copy(tab.at[idxv.at[j]], acc, sem, add=True)
                 for j in range(1, k)]
        for d in descs:
            d.wait()
        pltpu.sync_copy(acc, out.at[pl.ds(wid * _SW, _SW)])
    return body


_color_kernels = {}


def _color_kernel(k, dout):
    key = (k, dout)
    if key not in _color_kernels:
        _color_kernels[key] = pl.kernel(
            _color_body(k),
            out_type=jax.ShapeDtypeStruct((_N, dout), jnp.float32),
            mesh=plsc.VectorSubcoreMesh(core_axis_name="c",
                                        subcore_axis_name="s",
                                        num_cores=_NC, num_subcores=_NS),
            scratch_types=[
                pltpu.VMEM((_SW, k), jnp.int32),
                pltpu.VMEM((k, _SW), jnp.int32),
                pltpu.VMEM((_SW, dout), jnp.float32),
                pltpu.SemaphoreType.DMA,
            ],
            compiler_params=pltpu.CompilerParams(use_tc_tiling_on_sc=False,
                                                 needs_layout_passes=False),
        )
    return _color_kernels[key]


def _fold_t_body(aff, factt, out):
    out[...] = (aff[...] + factt[...]).T


def _fold_table(aff, factt, fb):
    b, d = aff.shape
    return pl.pallas_call(
        _fold_t_body,
        grid=(d // fb,),
        in_specs=[pl.BlockSpec((b, fb), lambda j: (0, j)),
                  pl.BlockSpec((b, fb), lambda j: (0, 0))],
        out_specs=pl.BlockSpec((fb, b), lambda j: (j, 0)),
        out_shape=jax.ShapeDtypeStruct((d, b), jnp.float32),
    )(aff, factt)


_BN = 1024


def _head_body(pov, w, b, pw, pb, wbias, bbias, pwbias, pbbias,
               f0a, f0b, f0c, p0a, p0b, p0c, p1t, p1c, f1t, f1c,
               f2a, f2b, f2c, f3a, f3b, f3c, f3d, out):
    q = pov[...]
    r = 1.0 - q
    wv = w[...] + wbias[...]
    bv = b[...] + bbias[...]
    pwv = pw[...] + pwbias[...]
    pbv = pb[...] + pbbias[...]
    base1 = jnp.maximum(q * wv + r * bv, 0.0)
    base2 = jnp.maximum(q * bv + r * wv, 0.0)
    pb1 = jnp.maximum(q * pwv + r * pbv, 0.0)
    pb2 = jnp.maximum(q * pbv + r * pwv, 0.0)
    dot = functools.partial(jnp.dot, preferred_element_type=jnp.float32)
    p = jnp.maximum(dot(pb1, p0a[...]) + dot(pb2, p0b[...]) + p0c[...], 0.0)
    p = dot(p, p1t[...]) + p1c[...]
    x = jnp.maximum(dot(base1, f0a[...]) + dot(base2, f0b[...]) + f0c[...] + p,
                    0.0)
    y = jnp.maximum(dot(x, f1t[...]) + f1c[...], 0.0)
    z = jnp.maximum(dot(x, f2a[...]) + dot(y, f2b[...]) + f2c[...], 0.0)
    out[...] = dot(x, f3a[...]) + dot(y, f3b[...]) + dot(z, f3c[...]) + f3d[...]


def _row_spec(cols):
    return pl.BlockSpec((_BN, cols), lambda i: (i, 0))


def _full_spec(rows, cols):
    return pl.BlockSpec((rows, cols), lambda i: (0, 0))


def kernel(pov, white_idx, black_idx, p_white_idx, p_black_idx,
           w_aff_W, w_aff_b, w_fact_W, b_aff_W, b_aff_b, b_fact_W,
           pw_aff_W, pw_aff_b, pw_fact_W, pb_aff_W, pb_aff_b, pb_fact_W,
           fc0_W, fc0_b, p_fc0_W, p_fc0_b, p_fc1_W, p_fc1_b,
           fc1_W, fc1_b, fc2_W, fc2_b, fc3_W, fc3_b):
    pw_comb = _fold_table(pw_aff_W, jnp.tile(pw_fact_W.T, (1, 4)), 4 * _F_PAWN)
    pb_comb = _fold_table(pb_aff_W, jnp.tile(pb_fact_W.T, (1, 4)), 4 * _F_PAWN)
    w_comb = _fold_table(w_aff_W, w_fact_W.T, _F_MAIN)

    pwsum = _color_kernel(_KP, _P_BASE)(p_white_idx, pw_comb)
    pbsum = _color_kernel(_KP, _P_BASE)(p_black_idx, pb_comb)
    eps = (pwsum[0, 0] + pbsum[0, 0]) * 0.0
    b_comb = _fold_table(b_aff_W, b_fact_W.T + eps, _F_MAIN)

    wsum = _color_kernel(_KM, _BASE)(white_idx, w_comb)
    bsum = _color_kernel(_KM, _BASE)(black_idx, b_comb)

    grid = (_N // _BN,)
    out = pl.pallas_call(
        _head_body,
        grid=grid,
        in_specs=[
            _row_spec(1),
            _row_spec(_BASE), _row_spec(_BASE),
            _row_spec(_P_BASE), _row_spec(_P_BASE),
            _full_spec(1, _BASE), _full_spec(1, _BASE),
            _full_spec(1, _P_BASE), _full_spec(1, _P_BASE),
            _full_spec(_BASE, 16), _full_spec(_BASE, 16), _full_spec(1, 16),
            _full_spec(_P_BASE, 16), _full_spec(_P_BASE, 16), _full_spec(1, 16),
            _full_spec(16, 16), _full_spec(1, 16),
            _full_spec(16, 16), _full_spec(1, 16),
            _full_spec(16, 16), _full_spec(16, 16), _full_spec(1, 16),
            _full_spec(16, 1), _full_spec(16, 1), _full_spec(16, 1),
            _full_spec(1, 1),
        ],
        out_specs=_row_spec(1),
        out_shape=jax.ShapeDtypeStruct((_N, 1), jnp.float32),
    )(
        pov, wsum, bsum, pwsum, pbsum,
        w_aff_b.reshape(1, -1), b_aff_b.reshape(1, -1),
        pw_aff_b.reshape(1, -1), pb_aff_b.reshape(1, -1),
        fc0_W[:, :_BASE].T, fc0_W[:, _BASE:].T, fc0_b.reshape(1, -1),
        p_fc0_W[:, :_P_BASE].T, p_fc0_W[:, _P_BASE:].T, p_fc0_b.reshape(1, -1),
        p_fc1_W.T, p_fc1_b.reshape(1, -1),
        fc1_W.T, fc1_b.reshape(1, -1),
        fc2_W[:, :16].T, fc2_W[:, 16:].T, fc2_b.reshape(1, -1),
        fc3_W[:, :16].T, fc3_W[:, 16:32].T, fc3_W[:, 32:].T,
        fc3_b.reshape(1, -1),
    )
    return out


# speedup vs baseline: 1.5686x; 1.2210x over previous
import functools

import jax
import jax.numpy as jnp
from jax import lax
from jax.experimental import pallas as pl
from jax.experimental.pallas import tpu as pltpu
from jax.experimental.pallas import tpu_sc as plsc

_D_MAIN = 49152
_D_PAWN = 8192
_F_MAIN = 768
_F_PAWN = 128
_BASE = 160
_P_BASE = 256
_N = 4096

_NC = 2
_NS = 16
_NW = _NC * _NS
_SW = _N // _NW
_KM = 32
_KP = 8


def _stage_idx_t(idx_hbm, blk, idxv, k, wid, lanes):
    pltpu.sync_copy(idx_hbm.at[pl.ds(wid * _SW, _SW)], blk)
    for g in range(_SW // 16):
        rows = lanes + g * 16
        for j in range(k):
            v = plsc.load_gather(blk, [rows, jnp.full((16,), j, jnp.int32)])
            idxv[j, pl.ds(g * 16, 16)] = v


def _color_body(k):
    def body(idx_hbm, tab, out, blk, idxv, acc, sem):
        wid = lax.axis_index("s") * _NC + lax.axis_index("c")
        lanes = lax.iota(jnp.int32, 16)
        _stage_idx_t(idx_hbm, blk, idxv, k, wid, lanes)
        pltpu.async_copy(tab.at[idxv.at[0]], acc, sem).wait()
        descs = [pltpu.async_copy(tab.at[idxv.at[j]], acc, sem, add=True)
                 for j in range(1, k)]
        for d in descs:
            d.wait()
        pltpu.sync_copy(acc, out.at[pl.ds(wid * _SW, _SW)])
    return body


def _main_body(idx_hbm, lo_t, hi_t, out, blk, idxv, acc_lo, acc_hi, sem):
    k = _KM
    wid = lax.axis_index("s") * _NC + lax.axis_index("c")
    lanes = lax.iota(jnp.int32, 16)
    _stage_idx_t(idx_hbm, blk, idxv, k, wid, lanes)
    d0 = pltpu.async_copy(lo_t.at[idxv.at[0]], acc_lo, sem)
    d1 = pltpu.async_copy(hi_t.at[idxv.at[0]], acc_hi, sem)
    d0.wait()
    d1.wait()
    descs = []
    for j in range(1, k):
        descs.append(pltpu.async_copy(lo_t.at[idxv.at[j]], acc_lo, sem,
                                      add=True))
        descs.append(pltpu.async_copy(hi_t.at[idxv.at[j]], acc_hi, sem,
                                      add=True))
    for d in descs:
        d.wait()
    rows = pl.ds(wid * _SW, _SW)
    pltpu.sync_copy(acc_lo, out.at[rows, pl.ds(0, 128)])
    pltpu.sync_copy(acc_hi, out.at[rows, pl.ds(128, 32)])


_sc_mesh_kw = dict(core_axis_name="c", subcore_axis_name="s",
                   num_cores=_NC, num_subcores=_NS)
_sc_params = None
_color_kernels = {}


def _sc_compiler_params():
    return pltpu.CompilerParams(use_tc_tiling_on_sc=False,
                                needs_layout_passes=False)


def _color_kernel(k, dout):
    key = (k, dout)
    if key not in _color_kernels:
        _color_kernels[key] = pl.kernel(
            _color_body(k),
            out_type=jax.ShapeDtypeStruct((_N, dout), jnp.float32),
            mesh=plsc.VectorSubcoreMesh(**_sc_mesh_kw),
            scratch_types=[
                pltpu.VMEM((_SW, k), jnp.int32),
                pltpu.VMEM((k, _SW), jnp.int32),
                pltpu.VMEM((_SW, dout), jnp.float32),
                pltpu.SemaphoreType.DMA,
            ],
            compiler_params=_sc_compiler_params(),
        )
    return _color_kernels[key]


def _main_kernel():
    key = "main"
    if key not in _color_kernels:
        _color_kernels[key] = pl.kernel(
            _main_body,
            out_type=jax.ShapeDtypeStruct((_N, _BASE), jnp.float32),
            mesh=plsc.VectorSubcoreMesh(**_sc_mesh_kw),
            scratch_types=[
                pltpu.VMEM((_SW, _KM), jnp.int32),
                pltpu.VMEM((_KM, _SW), jnp.int32),
                pltpu.VMEM((_SW, 128), jnp.float32),
                pltpu.VMEM((_SW, 32), jnp.float32),
                pltpu.SemaphoreType.DMA,
            ],
            compiler_params=_sc_compiler_params(),
        )
    return _color_kernels[key]


def _fold_t_body(aff, factt, out):
    out[...] = (aff[...] + factt[...]).T


def _fold_table(aff, factt, fb):
    b, d = aff.shape
    return pl.pallas_call(
        _fold_t_body,
        grid=(d // fb,),
        in_specs=[pl.BlockSpec((b, fb), lambda j: (0, j)),
                  pl.BlockSpec((b, fb), lambda j: (0, 0))],
        out_specs=pl.BlockSpec((fb, b), lambda j: (j, 0)),
        out_shape=jax.ShapeDtypeStruct((d, b), jnp.float32),
    )(aff, factt)


def _fold_split_body(aff, factt, lo, hi):
    t = (aff[...] + factt[...]).T
    lo[...] = t[:, :128]
    hi[...] = t[:, 128:]


def _fold_table_split(aff, factt, fb):
    b, d = aff.shape
    return pl.pallas_call(
        _fold_split_body,
        grid=(d // fb,),
        in_specs=[pl.BlockSpec((b, fb), lambda j: (0, j)),
                  pl.BlockSpec((b, fb), lambda j: (0, 0))],
        out_specs=[pl.BlockSpec((fb, 128), lambda j: (j, 0)),
                   pl.BlockSpec((fb, 32), lambda j: (j, 0))],
        out_shape=[jax.ShapeDtypeStruct((d, 128), jnp.float32),
                   jax.ShapeDtypeStruct((d, 32), jnp.float32)],
    )(aff, factt)


_BN = 1024


def _head_body(pov, w, b, pw, pb, wbias, bbias, pwbias, pbbias,
               f0a, f0b, f0c, p0a, p0b, p0c, p1t, p1c, f1t, f1c,
               f2a, f2b, f2c, f3a, f3b, f3c, f3d, out):
    q = pov[...]
    r = 1.0 - q
    wv = w[...] + wbias[...]
    bv = b[...] + bbias[...]
    pwv = pw[...] + pwbias[...]
    pbv = pb[...] + pbbias[...]
    base1 = jnp.maximum(q * wv + r * bv, 0.0)
    base2 = jnp.maximum(q * bv + r * wv, 0.0)
    pb1 = jnp.maximum(q * pwv + r * pbv, 0.0)
    pb2 = jnp.maximum(q * pbv + r * pwv, 0.0)
    dot = functools.partial(jnp.dot, preferred_element_type=jnp.float32)
    p = jnp.maximum(dot(pb1, p0a[...]) + dot(pb2, p0b[...]) + p0c[...], 0.0)
    p = dot(p, p1t[...]) + p1c[...]
    x = jnp.maximum(dot(base1, f0a[...]) + dot(base2, f0b[...]) + f0c[...] + p,
                    0.0)
    y = jnp.maximum(dot(x, f1t[...]) + f1c[...], 0.0)
    z = jnp.maximum(dot(x, f2a[...]) + dot(y, f2b[...]) + f2c[...], 0.0)
    out[...] = dot(x, f3a[...]) + dot(y, f3b[...]) + dot(z, f3c[...]) + f3d[...]


def _row_spec(cols):
    return pl.BlockSpec((_BN, cols), lambda i: (i, 0))


def _full_spec(rows, cols):
    return pl.BlockSpec((rows, cols), lambda i: (0, 0))


def kernel(pov, white_idx, black_idx, p_white_idx, p_black_idx,
           w_aff_W, w_aff_b, w_fact_W, b_aff_W, b_aff_b, b_fact_W,
           pw_aff_W, pw_aff_b, pw_fact_W, pb_aff_W, pb_aff_b, pb_fact_W,
           fc0_W, fc0_b, p_fc0_W, p_fc0_b, p_fc1_W, p_fc1_b,
           fc1_W, fc1_b, fc2_W, fc2_b, fc3_W, fc3_b):
    pw_comb = _fold_table(pw_aff_W, jnp.tile(pw_fact_W.T, (1, 4)), 4 * _F_PAWN)
    pb_comb = _fold_table(pb_aff_W, jnp.tile(pb_fact_W.T, (1, 4)), 4 * _F_PAWN)
    w_lo, w_hi = _fold_table_split(w_aff_W, w_fact_W.T, _F_MAIN)

    pwsum = _color_kernel(_KP, _P_BASE)(p_white_idx, pw_comb)
    pbsum = _color_kernel(_KP, _P_BASE)(p_black_idx, pb_comb)
    eps = (pwsum[0, 0] + pbsum[0, 0]) * 0.0
    b_lo, b_hi = _fold_table_split(b_aff_W, b_fact_W.T + eps, _F_MAIN)

    wsum = _main_kernel()(white_idx, w_lo, w_hi)
    bsum = _main_kernel()(black_idx, b_lo, b_hi)

    grid = (_N // _BN,)
    out = pl.pallas_call(
        _head_body,
        grid=grid,
        in_specs=[
            _row_spec(1),
            _row_spec(_BASE), _row_spec(_BASE),
            _row_spec(_P_BASE), _row_spec(_P_BASE),
            _full_spec(1, _BASE), _full_spec(1, _BASE),
            _full_spec(1, _P_BASE), _full_spec(1, _P_BASE),
            _full_spec(_BASE, 16), _full_spec(_BASE, 16), _full_spec(1, 16),
            _full_spec(_P_BASE, 16), _full_spec(_P_BASE, 16), _full_spec(1, 16),
            _full_spec(16, 16), _full_spec(1, 16),
            _full_spec(16, 16), _full_spec(1, 16),
            _full_spec(16, 16), _full_spec(16, 16), _full_spec(1, 16),
            _full_spec(16, 1), _full_spec(16, 1), _full_spec(16, 1),
            _full_spec(1, 1),
        ],
        out_specs=_row_spec(1),
        out_shape=jax.ShapeDtypeStruct((_N, 1), jnp.float32),
    )(
        pov, wsum, bsum, pwsum, pbsum,
        w_aff_b.reshape(1, -1), b_aff_b.reshape(1, -1),
        pw_aff_b.reshape(1, -1), pb_aff_b.reshape(1, -1),
        fc0_W[:, :_BASE].T, fc0_W[:, _BASE:].T, fc0_b.reshape(1, -1),
        p_fc0_W[:, :_P_BASE].T, p_fc0_W[:, _P_BASE:].T, p_fc0_b.reshape(1, -1),
        p_fc1_W.T, p_fc1_b.reshape(1, -1),
        fc1_W.T, fc1_b.reshape(1, -1),
        fc2_W[:, :16].T, fc2_W[:, 16:].T, fc2_b.reshape(1, -1),
        fc3_W[:, :16].T, fc3_W[:, 16:32].T, fc3_W[:, 32:].T,
        fc3_b.reshape(1, -1),
    )
    return out


# speedup vs baseline: 1.7051x; 1.0870x over previous
import functools

import jax
import jax.numpy as jnp
from jax import lax
from jax.experimental import pallas as pl
from jax.experimental.pallas import tpu as pltpu
from jax.experimental.pallas import tpu_sc as plsc

_D_MAIN = 49152
_D_PAWN = 8192
_F_MAIN = 768
_F_PAWN = 128
_BASE = 160
_P_BASE = 256
_N = 4096

_NC = 2
_NS = 16
_NW = _NC * _NS
_SW = _N // _NW
_KM = 32
_KP = 8


def _stage_idx_t(idx_hbm, blk, idxv, k, wid, lanes):
    pltpu.sync_copy(idx_hbm.at[pl.ds(wid * k, k)], blk)
    for g in range(_SW // 16):
        pos0 = (lanes + g * 16) * k
        for j in range(k):
            pos = pos0 + j
            v = plsc.load_gather(
                blk, [jnp.right_shift(pos, 7), jnp.bitwise_and(pos, 127)])
            idxv[j, pl.ds(g * 16, 16)] = v


def _color_body(k):
    def body(idx_hbm, tab, out, blk, idxv, acc, sem):
        wid = lax.axis_index("s") * _NC + lax.axis_index("c")
        lanes = lax.iota(jnp.int32, 16)
        _stage_idx_t(idx_hbm, blk, idxv, k, wid, lanes)
        pltpu.async_copy(tab.at[idxv.at[0]], acc, sem).wait()
        descs = [pltpu.async_copy(tab.at[idxv.at[j]], acc, sem, add=True)
                 for j in range(1, k)]
        for d in descs:
            d.wait()
        pltpu.sync_copy(acc, out.at[pl.ds(wid * _SW, _SW)])
    return body


def _main_body(idx_hbm, lo_t, hi_t, out, blk, idxv, acc_lo, acc_hi, sem):
    k = _KM
    wid = lax.axis_index("s") * _NC + lax.axis_index("c")
    lanes = lax.iota(jnp.int32, 16)
    _stage_idx_t(idx_hbm, blk, idxv, k, wid, lanes)
    d0 = pltpu.async_copy(lo_t.at[idxv.at[0]], acc_lo, sem)
    d1 = pltpu.async_copy(hi_t.at[idxv.at[0]], acc_hi, sem)
    d0.wait()
    d1.wait()
    descs = []
    for j in range(1, k):
        descs.append(pltpu.async_copy(lo_t.at[idxv.at[j]], acc_lo, sem,
                                      add=True))
        descs.append(pltpu.async_copy(hi_t.at[idxv.at[j]], acc_hi, sem,
                                      add=True))
    for d in descs:
        d.wait()
    rows = pl.ds(wid * _SW, _SW)
    pltpu.sync_copy(acc_lo, out.at[rows, pl.ds(0, 128)])
    pltpu.sync_copy(acc_hi, out.at[rows, pl.ds(128, 32)])


_sc_mesh_kw = dict(core_axis_name="c", subcore_axis_name="s",
                   num_cores=_NC, num_subcores=_NS)
_sc_params = None
_color_kernels = {}


def _sc_compiler_params():
    return pltpu.CompilerParams(use_tc_tiling_on_sc=False,
                                needs_layout_passes=False)


def _color_kernel(k, dout):
    key = (k, dout)
    if key not in _color_kernels:
        _color_kernels[key] = pl.kernel(
            _color_body(k),
            out_type=jax.ShapeDtypeStruct((_N, dout), jnp.float32),
            mesh=plsc.VectorSubcoreMesh(**_sc_mesh_kw),
            scratch_types=[
                pltpu.VMEM((k, 128), jnp.int32),
                pltpu.VMEM((k, _SW), jnp.int32),
                pltpu.VMEM((_SW, dout), jnp.float32),
                pltpu.SemaphoreType.DMA,
            ],
            compiler_params=_sc_compiler_params(),
        )
    return _color_kernels[key]


def _main_kernel():
    key = "main"
    if key not in _color_kernels:
        _color_kernels[key] = pl.kernel(
            _main_body,
            out_type=jax.ShapeDtypeStruct((_N, _BASE), jnp.float32),
            mesh=plsc.VectorSubcoreMesh(**_sc_mesh_kw),
            scratch_types=[
                pltpu.VMEM((_KM, 128), jnp.int32),
                pltpu.VMEM((_KM, _SW), jnp.int32),
                pltpu.VMEM((_SW, 128), jnp.float32),
                pltpu.VMEM((_SW, 32), jnp.float32),
                pltpu.SemaphoreType.DMA,
            ],
            compiler_params=_sc_compiler_params(),
        )
    return _color_kernels[key]


def _fold_t_body(aff, factt, out):
    out[...] = (aff[...] + factt[...]).T


def _fold_table(aff, factt, fb):
    b, d = aff.shape
    return pl.pallas_call(
        _fold_t_body,
        grid=(d // fb,),
        in_specs=[pl.BlockSpec((b, fb), lambda j: (0, j)),
                  pl.BlockSpec((b, fb), lambda j: (0, 0))],
        out_specs=pl.BlockSpec((fb, b), lambda j: (j, 0)),
        out_shape=jax.ShapeDtypeStruct((d, b), jnp.float32),
    )(aff, factt)


def _fold_split_body(aff, factt, lo, hi):
    t = (aff[...] + factt[...]).T
    lo[...] = t[:, :128]
    hi[...] = t[:, 128:]


def _fold_table_split(aff, factt, fb):
    b, d = aff.shape
    return pl.pallas_call(
        _fold_split_body,
        grid=(d // fb,),
        in_specs=[pl.BlockSpec((b, fb), lambda j: (0, j)),
                  pl.BlockSpec((b, fb), lambda j: (0, 0))],
        out_specs=[pl.BlockSpec((fb, 128), lambda j: (j, 0)),
                   pl.BlockSpec((fb, 32), lambda j: (j, 0))],
        out_shape=[jax.ShapeDtypeStruct((d, 128), jnp.float32),
                   jax.ShapeDtypeStruct((d, 32), jnp.float32)],
    )(aff, factt)


_BN = 1024


def _head_body(pov, w, b, pw, pb, wbias, bbias, pwbias, pbbias,
               f0a, f0b, f0c, p0a, p0b, p0c, p1t, p1c, f1t, f1c,
               f2a, f2b, f2c, f3a, f3b, f3c, f3d, out):
    q = pov[...]
    r = 1.0 - q
    wv = w[...] + wbias[...]
    bv = b[...] + bbias[...]
    pwv = pw[...] + pwbias[...]
    pbv = pb[...] + pbbias[...]
    base1 = jnp.maximum(q * wv + r * bv, 0.0)
    base2 = jnp.maximum(q * bv + r * wv, 0.0)
    pb1 = jnp.maximum(q * pwv + r * pbv, 0.0)
    pb2 = jnp.maximum(q * pbv + r * pwv, 0.0)
    dot = functools.partial(jnp.dot, preferred_element_type=jnp.float32)
    p = jnp.maximum(dot(pb1, p0a[...]) + dot(pb2, p0b[...]) + p0c[...], 0.0)
    p = dot(p, p1t[...]) + p1c[...]
    x = jnp.maximum(dot(base1, f0a[...]) + dot(base2, f0b[...]) + f0c[...] + p,
                    0.0)
    y = jnp.maximum(dot(x, f1t[...]) + f1c[...], 0.0)
    z = jnp.maximum(dot(x, f2a[...]) + dot(y, f2b[...]) + f2c[...], 0.0)
    out[...] = dot(x, f3a[...]) + dot(y, f3b[...]) + dot(z, f3c[...]) + f3d[...]


def _row_spec(cols):
    return pl.BlockSpec((_BN, cols), lambda i: (i, 0))


def _full_spec(rows, cols):
    return pl.BlockSpec((rows, cols), lambda i: (0, 0))


def kernel(pov, white_idx, black_idx, p_white_idx, p_black_idx,
           w_aff_W, w_aff_b, w_fact_W, b_aff_W, b_aff_b, b_fact_W,
           pw_aff_W, pw_aff_b, pw_fact_W, pb_aff_W, pb_aff_b, pb_fact_W,
           fc0_W, fc0_b, p_fc0_W, p_fc0_b, p_fc1_W, p_fc1_b,
           fc1_W, fc1_b, fc2_W, fc2_b, fc3_W, fc3_b):
    pw_comb = _fold_table(pw_aff_W, jnp.tile(pw_fact_W.T, (1, 4)), 4 * _F_PAWN)
    pb_comb = _fold_table(pb_aff_W, jnp.tile(pb_fact_W.T, (1, 4)), 4 * _F_PAWN)
    w_lo, w_hi = _fold_table_split(w_aff_W, jnp.tile(w_fact_W.T, (1, 2)),
                                   2 * _F_MAIN)

    pwsum = _color_kernel(_KP, _P_BASE)(
        p_white_idx.reshape(_N * _KP // 128, 128), pw_comb)
    pbsum = _color_kernel(_KP, _P_BASE)(
        p_black_idx.reshape(_N * _KP // 128, 128), pb_comb)
    eps = (pwsum[0, 0] + pbsum[0, 0]) * 0.0
    b_lo, b_hi = _fold_table_split(b_aff_W, jnp.tile(b_fact_W.T, (1, 2)) + eps,
                                   2 * _F_MAIN)

    wsum = _main_kernel()(white_idx.reshape(_N * _KM // 128, 128), w_lo, w_hi)
    bsum = _main_kernel()(black_idx.reshape(_N * _KM // 128, 128), b_lo, b_hi)

    grid = (_N // _BN,)
    out = pl.pallas_call(
        _head_body,
        grid=grid,
        in_specs=[
            _row_spec(1),
            _row_spec(_BASE), _row_spec(_BASE),
            _row_spec(_P_BASE), _row_spec(_P_BASE),
            _full_spec(1, _BASE), _full_spec(1, _BASE),
            _full_spec(1, _P_BASE), _full_spec(1, _P_BASE),
            _full_spec(_BASE, 16), _full_spec(_BASE, 16), _full_spec(1, 16),
            _full_spec(_P_BASE, 16), _full_spec(_P_BASE, 16), _full_spec(1, 16),
            _full_spec(16, 16), _full_spec(1, 16),
            _full_spec(16, 16), _full_spec(1, 16),
            _full_spec(16, 16), _full_spec(16, 16), _full_spec(1, 16),
            _full_spec(16, 1), _full_spec(16, 1), _full_spec(16, 1),
            _full_spec(1, 1),
        ],
        out_specs=_row_spec(1),
        out_shape=jax.ShapeDtypeStruct((_N, 1), jnp.float32),
    )(
        pov, wsum, bsum, pwsum, pbsum,
        w_aff_b.reshape(1, -1), b_aff_b.reshape(1, -1),
        pw_aff_b.reshape(1, -1), pb_aff_b.reshape(1, -1),
        fc0_W[:, :_BASE].T, fc0_W[:, _BASE:].T, fc0_b.reshape(1, -1),
        p_fc0_W[:, :_P_BASE].T, p_fc0_W[:, _P_BASE:].T, p_fc0_b.reshape(1, -1),
        p_fc1_W.T, p_fc1_b.reshape(1, -1),
        fc1_W.T, fc1_b.reshape(1, -1),
        fc2_W[:, :16].T, fc2_W[:, 16:].T, fc2_b.reshape(1, -1),
        fc3_W[:, :16].T, fc3_W[:, 16:32].T, fc3_W[:, 32:].T,
        fc3_b.reshape(1, -1),
    )
    return out
